# baseline (speedup 1.0000x reference)
import functools

import jax
import jax.numpy as jnp
from jax import lax
from jax.experimental import pallas as pl
from jax.experimental.pallas import tpu as pltpu
from jax.experimental.pallas import tpu_sc as plsc

B = 16384
D = 64
N_ITEMS = 1000000
L = 16
NC = 2
NS = 16
NW = NC * NS
CH = 512
NCH = 61
IPW = CH * NCH
TAIL0 = NW * IPW
TAILW = N_ITEMS - TAIL0
TAILB0 = N_ITEMS - 128
SLABW = CH + 11
HITCAP = 4096
WLCAP = 2048
DUMP = B
BPW = B // NW
PIECE = 4096


def _stage_body(u_hbm, i_hbm, tt_hbm, tail_hbm, gu_hbm, gi_hbm,
                slab0, slab1, pbuf, hits, wl, stage, bu_v, bi_v,
                s0, s1, sp):
    wid = lax.axis_index("s") * NC + lax.axis_index("c")
    start = wid * IPW
    is_last = wid == NW - 1
    rw = jnp.where(is_last, IPW + TAILW, IPW)

    iota = lax.iota(jnp.int32, L)
    zero16 = jnp.zeros((L,), jnp.int32)

    def scan_piece(cnt, src_hbm, piece, flag):
        pltpu.sync_copy(src_hbm.at[pl.ds(piece * PIECE, PIECE)], pbuf)
        bbase = piece * PIECE

        def vec_step(v, cnt):
            idx = pbuf[pl.ds(v * L, L)]
            lo = idx - start
            m = (lo >= 0) & (lo < rw)
            bpos = bbase + v * L + iota
            packed = bpos | (lo << 14) | (flag << 29)
            mi = jnp.where(m, 1, 0)
            ranks = plsc.cumsum(mi) - mi
            pos = jnp.minimum(cnt + ranks, HITCAP - 1)
            plsc.store_scatter(hits, [pos], packed, mask=m)
            return cnt + plsc.all_reduce_population_count(m)

        return lax.fori_loop(0, PIECE // L, vec_step, cnt)

    cnt = zero16
    for piece in range(B // PIECE):
        cnt = scan_piece(cnt, u_hbm, piece, 0)
    for piece in range(B // PIECE):
        cnt = scan_piece(cnt, i_hbm, piece, 1)
    cnt = jnp.minimum(cnt, HITCAP)
    n_hit_vec = (cnt[0] + (L - 1)) // L

    slabs = (slab0, slab1)
    sems = (s0, s1)

    def issue(ch, par):
        off = pl.multiple_of(start + ch * CH, 128)
        pltpu.async_copy(
            tt_hbm.at[:, pl.ds(off, CH)], slabs[par].at[:, pl.ds(0, CH)],
            sems[par])

    def wait(ch, par):
        off = pl.multiple_of(start + ch * CH, 128)
        pltpu.make_async_copy(
            tt_hbm.at[:, pl.ds(off, CH)], slabs[par].at[:, pl.ds(0, CH)],
            sems[par]).wait()

    def process(rel, par, w):
        slab = slabs[par]

        def match_vec(v, wcnt):
            e = hits[pl.ds(v * L, L)]
            lo = jnp.right_shift(e, 14) & 0x7FFF
            valid = (v * L + iota) < cnt
            m = valid & (lo >= rel) & (lo < rel + w)
            mi = jnp.where(m, 1, 0)
            ranks = plsc.cumsum(mi) - mi
            pos = jnp.minimum(wcnt + ranks, WLCAP - 1)
            plsc.store_scatter(wl, [pos], e, mask=m)
            return wcnt + plsc.all_reduce_population_count(m)

        wcnt = lax.fori_loop(0, n_hit_vec, match_vec, zero16)
        n_groups = (wcnt[0] + (L - 1)) // L

        def group_step(g, _):
            e = wl[pl.ds(g * L, L)]
            lo = jnp.right_shift(e, 14) & 0x7FFF
            bpos = e & 0x3FFF
            flag = jnp.right_shift(e, 29) & 1
            gvalid = (g * L + iota) < wcnt
            loc = jnp.clip(lo - rel, 0, w - 1)
            for c in range(D):
                cvec = jnp.full((L,), c, jnp.int32)
                vc = plsc.load_gather(slab, [cvec, loc])
                plsc.store_scatter(stage, [iota, cvec], vc)
            bu = jnp.where(gvalid & (flag == 0), bpos, DUMP)
            bi = jnp.where(gvalid & (flag == 1), bpos, DUMP)
            bu_v[...] = bu
            bi_v[...] = bi
            cu = pltpu.async_copy(stage.at[:, pl.ds(0, 2 * D)],
                                  gu_hbm.at[bu_v], sp)
            cu.wait()
            ci = pltpu.async_copy(stage.at[:, pl.ds(0, 2 * D)],
                                  gi_hbm.at[bi_v], sp)
            ci.wait()
            return 0

        lax.fori_loop(0, n_groups, group_step, 0)

    issue(0, 0)
    issue(1, 1)

    def chunk_iter(it, _):
        for par in range(2):
            ch = it * 2 + par
            wait(ch, par)
            process(ch * CH, par, CH)

            @pl.when(ch + 2 < NCH)
            def _():
                issue(ch + 2, par)
        return 0

    lax.fori_loop(0, (NCH - 1) // 2, chunk_iter, 0)
    wait(NCH - 1, 0)
    process((NCH - 1) * CH, 0, CH)

    @pl.when(is_last)
    def _():
        off1 = pl.multiple_of(TAIL0, 128)
        pltpu.sync_copy(tt_hbm.at[:, pl.ds(off1, CH)],
                        slabs[0].at[:, pl.ds(0, CH)])
        process(NCH * CH, 0, CH)
        pltpu.sync_copy(tail_hbm, slabs[1].at[:, pl.ds(0, 128)])
        process(TAILB0 - (NW - 1) * IPW, 1, 128)


TR = 17


HB = BPW // 2


def _combine_body(gu_hbm, gi_hbm, w_hbm, b_hbm, out_hbm,
                  gu_v, gi_v, w_v, b_v, tr, out_v):
    wid = lax.axis_index("s") * NC + lax.axis_index("c")
    base = wid * BPW
    pltpu.sync_copy(w_hbm.at[0], w_v)
    pltpu.sync_copy(b_hbm, b_v)

    w_chunks = [w_v[pl.ds(k * L, L)] for k in range(D // L)]
    b_vec = b_v[...]
    lane = lax.iota(jnp.int32, L)

    def half(h, _):
        hbase = base + h * HB
        pltpu.sync_copy(gu_hbm.at[pl.ds(hbase, HB)], gu_v)
        pltpu.sync_copy(gi_hbm.at[pl.ds(hbase, HB)], gi_v)

        def group(g, _):
            for j in range(L):
                e = g * L + j
                p = jnp.zeros((L,), jnp.float32)
                for k in range(D // L):
                    pu = gu_v[e, pl.ds(k * L, L)]
                    pi = gi_v[e, pl.ds(k * L, L)]
                    p = p + (pu * pi) * w_chunks[k]
                plsc.store_scatter(tr, [lane * TR + j], p)
            acc = b_vec
            for k in range(L):
                acc = acc + tr[pl.ds(k * TR, L)]
            out_v[pl.ds(g * L, L)] = 1.0 / (1.0 + jnp.exp(-acc))
            return 0

        lax.fori_loop(0, HB // L, group, 0)
        pltpu.sync_copy(out_v, out_hbm.at[pl.ds(hbase, HB)])
        return 0

    lax.fori_loop(0, 2, half, 0)


@jax.jit
def _gmf(u_input, i_input, table_t, tail_t, W, b16):
    mesh = plsc.VectorSubcoreMesh(core_axis_name="c", subcore_axis_name="s")
    stage_fn = functools.partial(
        pl.kernel,
        mesh=mesh,
        compiler_params=pltpu.CompilerParams(needs_layout_passes=False),
        out_type=(jax.ShapeDtypeStruct((B + 1, 2 * D), jnp.float32),
                  jax.ShapeDtypeStruct((B + 1, 2 * D), jnp.float32)),
        scratch_types=[
            pltpu.VMEM((D, SLABW), jnp.float32),
            pltpu.VMEM((D, SLABW), jnp.float32),
            pltpu.VMEM((PIECE,), jnp.int32),
            pltpu.VMEM((HITCAP,), jnp.int32),
            pltpu.VMEM((WLCAP,), jnp.int32),
            pltpu.VMEM((L, 2 * D + 1), jnp.float32),
            pltpu.VMEM((L,), jnp.int32),
            pltpu.VMEM((L,), jnp.int32),
            pltpu.SemaphoreType.DMA,
            pltpu.SemaphoreType.DMA,
            pltpu.SemaphoreType.DMA,
        ],
    )(_stage_body)
    gu, gi = stage_fn(u_input, i_input, table_t, tail_t)

    combine_fn = functools.partial(
        pl.kernel,
        mesh=mesh,
        compiler_params=pltpu.CompilerParams(needs_layout_passes=False),
        out_type=jax.ShapeDtypeStruct((B,), jnp.float32),
        scratch_types=[
            pltpu.VMEM((BPW // 2, 2 * D), jnp.float32),
            pltpu.VMEM((BPW // 2, 2 * D), jnp.float32),
            pltpu.VMEM((D,), jnp.float32),
            pltpu.VMEM((L,), jnp.float32),
            pltpu.VMEM((L * TR,), jnp.float32),
            pltpu.VMEM((BPW // 2,), jnp.float32),
        ],
    )(_combine_body)
    return combine_fn(gu, gi, W, b16)


def kernel(u_input, i_input, item_table, W, b):
    u32 = u_input.astype(jnp.int32)
    i32 = i_input.astype(jnp.int32)
    b16 = jnp.broadcast_to(b.astype(jnp.float32), (L,))
    tail_t = item_table.T[:, TAILB0:]
    return _gmf(u32, i32, item_table.T, tail_t, W, b16)
